# baseline (speedup 1.0000x reference)
import functools
import math

import jax
import jax.numpy as jnp
import numpy as np
from jax import lax
from jax.experimental import pallas as pl
from jax.experimental.pallas import tpu as pltpu
from jax.experimental.pallas import tpu_sc as plsc

D_MODEL = 64
SEQ = 200
BATCH = 4096
SPLIT_A = 104
SPLIT_B = SEQ - SPLIT_A
NBUF = 4
LANES = 16

_info = plsc.get_sparse_core_info()
NC, NS = _info.num_cores, _info.num_subcores
NW = NC * NS
SEQ_PER_W = BATCH // NW


def _positional_encoding() -> np.ndarray:
    position = np.arange(0, SEQ, dtype=np.float32)[:, None]
    div_term = np.exp(
        np.arange(0, D_MODEL, 2, dtype=np.float32) * (-math.log(10000.0) / D_MODEL)
    )
    pe = np.zeros((SEQ, D_MODEL), dtype=np.float32)
    pe[:, 0::2] = np.sin(position * div_term)
    pe[:, 1::2] = np.cos(position * div_term)
    return pe


_PE = _positional_encoding()

_mesh = plsc.VectorSubcoreMesh(core_axis_name="c", subcore_axis_name="s")


@functools.partial(
    pl.kernel,
    out_type=jax.ShapeDtypeStruct((BATCH, SEQ, D_MODEL), jnp.float32),
    mesh=_mesh,
    compiler_params=pltpu.CompilerParams(use_tc_tiling_on_sc=False),
    scratch_types=(
        [
            pltpu.VMEM((SEQ_PER_W, SPLIT_A), jnp.int32),
            pltpu.VMEM((SEQ_PER_W, SPLIT_B), jnp.int32),
            pltpu.VMEM((SEQ, D_MODEL), jnp.float32),
        ]
        + [pltpu.VMEM((SEQ, D_MODEL), jnp.float32) for _ in range(NBUF)]
        + [pltpu.SemaphoreType.DMA for _ in range(2 * NBUF)]
    ),
)
def _emb_kernel(
    table_hbm,
    xa_hbm,
    xb_hbm,
    pe_hbm,
    out_hbm,
    idx_a,
    idx_b,
    pe_v,
    buf0,
    buf1,
    buf2,
    buf3,
    g0,
    g1,
    g2,
    g3,
    s0,
    s1,
    s2,
    s3,
):
    bufs = (buf0, buf1, buf2, buf3)
    gsems = (g0, g1, g2, g3)
    ssems = (s0, s1, s2, s3)
    wid = lax.axis_index("s") * NC + lax.axis_index("c")
    seq0 = wid * SEQ_PER_W

    pltpu.sync_copy(xa_hbm.at[pl.ds(seq0, SEQ_PER_W)], idx_a)
    pltpu.sync_copy(xb_hbm.at[pl.ds(seq0, SEQ_PER_W)], idx_b)
    pltpu.sync_copy(pe_hbm, pe_v)

    def gather_descs(t, b):
        return (
            pltpu.make_async_copy(
                table_hbm.at[idx_a.at[t]], bufs[b].at[pl.ds(0, SPLIT_A)], gsems[b]
            ),
            pltpu.make_async_copy(
                table_hbm.at[idx_b.at[t]], bufs[b].at[pl.ds(SPLIT_A, SPLIT_B)], gsems[b]
            ),
        )

    def store_desc(t, b):
        return pltpu.make_async_copy(bufs[b], out_hbm.at[seq0 + t], ssems[b])

    def start_gather(t, b):
        for d in gather_descs(t, b):
            d.start()

    def wait_gather(t, b):
        for d in gather_descs(t, b):
            d.wait()

    for b in range(NBUF):
        start_gather(b, b)

    def outer(i, carry):
        t0 = i * NBUF
        for b in range(NBUF):
            t = t0 + b
            wait_gather(t, b)
            nxt = t + 1
            fb = (b + 1) % NBUF

            @pl.when(jnp.logical_and(nxt >= NBUF, nxt < SEQ_PER_W))
            def _():
                store_desc(nxt - NBUF, fb).wait()
                start_gather(nxt, fb)

            def add_pe(r, c2):
                for c in range(D_MODEL // LANES):
                    sl = pl.ds(c * LANES, LANES)
                    bufs[b][r, sl] = bufs[b][r, sl] + pe_v[r, sl]
                return c2

            lax.fori_loop(0, SEQ, add_pe, 0, unroll=2)
            store_desc(t, b).start()
        return carry

    lax.fori_loop(0, SEQ_PER_W // NBUF, outer, 0)

    for b in range(NBUF):
        store_desc(SEQ_PER_W - NBUF + b, b).wait()


def kernel(x, table):
    xa = x[:, :SPLIT_A]
    xb = x[:, SPLIT_A:]
    pe = jnp.asarray(_PE)
    return _emb_kernel(table, xa, xb, pe)
